# baseline (speedup 1.0000x reference)
import functools

import jax
import jax.numpy as jnp
from jax import lax
from jax.experimental import pallas as pl
from jax.experimental.pallas import tpu as pltpu
from jax.experimental.pallas import tpu_sc as plsc

B = 16384
F = 64
U = 1000000
I = 100000
_NC = 2
_NS = 16
_NW = _NC * _NS
_NV = B // 16
_UCOLS = U // 128 + 1
_ICOLS = I // 128 + 1
_UFULL = U // 128
_IFULL = I // 128
_UCPT = 248
_ICPT = 28
_CAP = 16
_NVC = 2048


def _sc_gather(user, item, tug_t, tig_t, tum_t, tim_t,
               tail_ug, tail_ig, tail_um, tail_im):
    mesh = plsc.VectorSubcoreMesh(core_axis_name="c", subcore_axis_name="s")

    @functools.partial(
        pl.kernel,
        mesh=mesh,
        out_type=[jax.ShapeDtypeStruct((B, F), jnp.float32)] * 4,
        scratch_types=[
            pltpu.VMEM((B + 16,), jnp.int32),
            pltpu.VMEM((B + 16,), jnp.int32),
            pltpu.VMEM((B + 16,), jnp.int32),
            pltpu.VMEM((272,), jnp.int32),
            pltpu.VMEM((272,), jnp.int32),
            pltpu.VMEM((272,), jnp.int32),
            pltpu.VMEM((_NVC + 16,), jnp.int32),
            pltpu.VMEM((F, 128), jnp.float32),
            pltpu.VMEM((F, 128), jnp.float32),
            pltpu.VMEM((F, 128), jnp.float32),
            pltpu.VMEM((F, 128), jnp.float32),
            pltpu.VMEM((F, 128), jnp.float32),
            pltpu.VMEM((F, 128), jnp.float32),
            pltpu.VMEM((F, 128), jnp.float32),
            pltpu.VMEM((F, 128), jnp.float32),
            pltpu.SemaphoreType.DMA,
            pltpu.SemaphoreType.DMA,
            pltpu.SemaphoreType.DMA,
            pltpu.SemaphoreType.DMA,
            pltpu.SemaphoreType.DMA,
            pltpu.SemaphoreType.DMA,
            pltpu.SemaphoreType.DMA,
            pltpu.SemaphoreType.DMA,
        ],
    )
    def k(user_h, item_h, tug_h, tig_h, tum_h, tim_h,
          tlug_h, tlig_h, tlum_h, tlim_h,
          oug_h, oig_h, oum_h, oim_h,
          idx_v, wl, swl, cnt, off, cur, vc,
          bufA0, bufA1, bufA2, bufA3, bufB0, bufB1, bufB2, bufB3,
          fs0, fs1, fs2, fs3, ws0, ws1, ws2, ws3):
        wid = lax.axis_index("s") * _NC + lax.axis_index("c")
        iota = lax.iota(jnp.int32, 16)
        zeros16 = jnp.zeros((16,), jnp.int32)
        bufsA = (bufA0, bufA1, bufA2, bufA3)
        bufsB = (bufB0, bufB1, bufB2, bufB3)
        fsems = (fs0, fs1, fs2, fs3)
        wsems = (ws0, ws1, ws2, ws3)

        def sread(ref, i):
            return ref[pl.ds(i, 16)][0]

        def swrite(ref, i, val):
            v = ref[pl.ds(i, 16)]
            ref[pl.ds(i, 16)] = jnp.where(iota == 0, val, v)

        def run_pass(idx_h, tabA_h, tabB_h, tailA_h, tailB_h,
                     outA_h, outB_h, total, full, ncpt):
            lo = jnp.minimum(wid * ncpt, total)
            hi = jnp.minimum(lo + ncpt, total)

            pltpu.sync_copy(idx_h, idx_v.at[pl.ds(0, B)])

            def scan_step(e, n):
                val = sread(idx_v, e)
                col = val >> 7
                inr = (col >= lo) & (col < hi)

                @pl.when(inr)
                def _():
                    swrite(wl, n,
                           ((col - lo) << 21) | ((val & 127) << 14) | e)

                return n + jnp.where(inr, 1, 0)

            n = lax.fori_loop(0, B, scan_step, jnp.int32(0))

            def zero_step(j, z):
                swrite(cnt, j, jnp.int32(0))
                return z

            lax.fori_loop(0, ncpt, zero_step, 0)

            def hist_step(e, z):
                rc = sread(wl, e) >> 21
                swrite(cnt, rc, sread(cnt, rc) + 1)
                return z

            lax.fori_loop(0, n, hist_step, 0)

            def pfx_step(j, acc):
                c = sread(cnt, j)
                swrite(off, j, acc)
                swrite(cur, j, acc)
                return acc + c

            lax.fori_loop(0, ncpt, pfx_step, jnp.int32(0))

            def sort_step(e, z):
                ent = sread(wl, e)
                rc = ent >> 21
                p = sread(cur, rc)
                swrite(swl, p, ent)
                swrite(cur, rc, p + 1)
                return z

            lax.fori_loop(0, n, sort_step, 0)

            swrite(vc, 0, jnp.int32(0))

            def vc_step(e, nvc):
                ent = sread(swl, e)
                rc = ent >> 21
                o = sread(off, rc)
                done = e - o
                newchunk = (done & (_CAP - 1)) == 0
                cc = jnp.minimum(sread(cnt, rc) - done, _CAP)

                @pl.when(newchunk)
                def _():
                    swrite(vc, nvc, (rc << 20) | (cc << 15) | e)

                return nvc + jnp.where(newchunk, 1, 0)

            nvc = lax.fori_loop(0, n, vc_step, jnp.int32(1))

            def vc_col(vi):
                d = sread(vc, jnp.minimum(vi, nvc - 1))
                return d >> 20

            def fire(vi, sp):
                jf = jnp.minimum(lo + vc_col(vi), total - 1)

                @pl.when(jf < full)
                def _():
                    pltpu.async_copy(tabA_h.at[:, pl.ds(jf * 128, 128)],
                                     bufsA[sp], fsems[sp])
                    pltpu.async_copy(tabB_h.at[:, pl.ds(jf * 128, 128)],
                                     bufsB[sp], fsems[sp])

                @pl.when(jf >= full)
                def _():
                    pltpu.async_copy(tailA_h.at[:, pl.ds(0, 128)],
                                     bufsA[sp], fsems[sp])
                    pltpu.async_copy(tailB_h.at[:, pl.ds(0, 128)],
                                     bufsB[sp], fsems[sp])

            def drain_fetch(sp):
                pltpu.make_async_copy(tabA_h.at[:, pl.ds(0, 128)],
                                      bufsA[sp], fsems[sp]).wait()
                pltpu.make_async_copy(tabA_h.at[:, pl.ds(0, 128)],
                                      bufsB[sp], fsems[sp]).wait()

            def drain_writes(sp, wn):
                for q in range(2 * _CAP):
                    @pl.when(q < wn)
                    def _():
                        pltpu.make_async_copy(bufsA[sp].at[:, 0],
                                              outA_h.at[0],
                                              wsems[sp]).wait()

            def process(vi, sp):
                d = sread(vc, jnp.minimum(vi, nvc - 1))
                cc = (d >> 15) & 31
                ss = d & 32767
                for i in range(_CAP):
                    @pl.when(i < cc)
                    def _():
                        ent = sread(swl, ss + i)
                        cv = (ent >> 14) & 127
                        pos = ent & 16383
                        pltpu.async_copy(bufsA[sp].at[:, cv],
                                         outA_h.at[pos], wsems[sp])
                        pltpu.async_copy(bufsB[sp].at[:, cv],
                                         outB_h.at[pos], wsems[sp])
                return 2 * cc

            fire(0, 0)
            fire(1, 1)

            def quad_step(tt, wns):
                wns = list(wns)
                vi0 = tt * 4
                for sp0 in range(4):
                    vi = vi0 + sp0
                    sp = sp0
                    spn = (sp0 + 2) & 3
                    drain_writes(spn, wns[spn])
                    wns[spn] = jnp.int32(0)
                    fire(vi + 2, spn)
                    drain_fetch(sp)
                    wns[sp] = process(vi, sp)
                return tuple(wns)

            nq = (nvc + 3) >> 2
            wns = lax.fori_loop(0, nq, quad_step, (jnp.int32(0),) * 4)
            for sp in range(4):
                drain_writes(sp, wns[sp])
            drain_fetch(0)
            drain_fetch(1)

        run_pass(user_h, tug_h, tum_h, tlug_h, tlum_h, oug_h, oum_h,
                 _UCOLS, _UFULL, _UCPT)
        run_pass(item_h, tig_h, tim_h, tlig_h, tlim_h, oig_h, oim_h,
                 _ICOLS, _IFULL, _ICPT)

    return k(user, item, tug_t, tig_t, tum_t, tim_t,
             tail_ug, tail_ig, tail_um, tail_im)


def _tc_head(eug, eig, eum, eim, w1a_t, w1b_t, b1r, wg, wh, bp11):
    BB = 2048

    def body(eug_r, eig_r, eum_r, eim_r, w1a_r, w1b_r, b1_r, wg_r, wh_r,
             bp_r, out_r):
        h = jnp.dot(eum_r[...], w1a_r[...], preferred_element_type=jnp.float32)
        h = h + jnp.dot(eim_r[...], w1b_r[...],
                        preferred_element_type=jnp.float32)
        h = jnp.maximum(h + b1_r[...], 0.0)
        gmf = eug_r[...] * eig_r[...]
        logit = (jnp.sum(gmf * wg_r[...], axis=1, keepdims=True)
                 + jnp.sum(h * wh_r[...], axis=1, keepdims=True)
                 + bp_r[...])
        out_r[...] = 1.0 / (1.0 + jnp.exp(-logit))

    batch_spec = pl.BlockSpec((BB, F), lambda i: (i, 0))
    full_spec = pl.BlockSpec((F, F), lambda i: (0, 0))
    row_spec = pl.BlockSpec((1, F), lambda i: (0, 0))
    return pl.pallas_call(
        body,
        grid=(B // BB,),
        in_specs=[batch_spec, batch_spec, batch_spec, batch_spec,
                  full_spec, full_spec, row_spec, row_spec, row_spec,
                  pl.BlockSpec((1, 1), lambda i: (0, 0))],
        out_specs=pl.BlockSpec((BB, 1), lambda i: (i, 0)),
        out_shape=jax.ShapeDtypeStruct((B, 1), jnp.float32),
    )(eug, eig, eum, eim, w1a_t, w1b_t, b1r, wg, wh, bp11)


def _tail128(tab_t, fullcols):
    t = tab_t[:, fullcols * 128:]
    return jnp.pad(t, ((0, 0), (0, 128 - t.shape[1])))


def kernel(user, item, embed_user_GMF, embed_item_GMF, embed_user_MLP,
           embed_item_MLP, W1, b1, Wp, bp):
    user = user.astype(jnp.int32)
    item = item.astype(jnp.int32)
    tug_t = embed_user_GMF.T
    tig_t = embed_item_GMF.T
    tum_t = embed_user_MLP.T
    tim_t = embed_item_MLP.T
    eug, eig, eum, eim = _sc_gather(
        user, item, tug_t, tig_t, tum_t, tim_t,
        _tail128(tug_t, _UFULL), _tail128(tig_t, _IFULL),
        _tail128(tum_t, _UFULL), _tail128(tim_t, _IFULL))
    w1a_t = W1[:, :F].T
    w1b_t = W1[:, F:].T
    b1r = b1.reshape(1, F)
    wg = Wp[0, :F].reshape(1, F)
    wh = Wp[0, F:].reshape(1, F)
    bp11 = bp.reshape(1, 1)
    out = _tc_head(eug, eig, eum, eim, w1a_t, w1b_t, b1r, wg, wh, bp11)
    return out.reshape(B)


# speedup vs baseline: 7.6648x; 7.6648x over previous
import functools

import jax
import jax.numpy as jnp
from jax import lax
from jax.experimental import pallas as pl
from jax.experimental.pallas import tpu as pltpu
from jax.experimental.pallas import tpu_sc as plsc

B = 16384
F = 64
_NC = 2
_NS = 16
_NW = _NC * _NS
_BPW = B // _NW
_CH = 128
_NCH = _BPW // _CH


def _sc_gather1(pidx, tab2):
    mesh = plsc.VectorSubcoreMesh(core_axis_name="c", subcore_axis_name="s")

    @functools.partial(
        pl.kernel,
        mesh=mesh,
        out_type=jax.ShapeDtypeStruct((B, 2 * F), jnp.float32),
        scratch_types=[
            pltpu.VMEM((_CH,), jnp.int32),
            pltpu.VMEM((_CH,), jnp.int32),
            pltpu.VMEM((_CH, 2 * F), jnp.float32),
            pltpu.VMEM((_CH, 2 * F), jnp.float32),
            pltpu.SemaphoreType.DMA,
            pltpu.SemaphoreType.DMA,
        ],
    )
    def k(pidx_h, tab_h, out_h, idx0, idx1, buf0, buf1, sem0, sem1):
        wid = lax.axis_index("s") * _NC + lax.axis_index("c")
        base = wid * _BPW
        idxs = (idx0, idx1)
        bufs = (buf0, buf1)
        sems = (sem0, sem1)

        def fire(c, sp):
            off = base + c * _CH
            pltpu.sync_copy(pidx_h.at[pl.ds(off, _CH)], idxs[sp])
            pltpu.async_copy(tab_h.at[idxs[sp]], bufs[sp], sems[sp])

        def drain_store(c, sp):
            pltpu.make_async_copy(tab_h.at[idxs[sp]], bufs[sp],
                                  sems[sp]).wait()
            off = base + c * _CH
            pltpu.sync_copy(bufs[sp], out_h.at[pl.ds(off, _CH)])

        fire(0, 0)

        def step(t, carry):
            c = t * 2

            @pl.when(c + 1 < _NCH)
            def _():
                fire(c + 1, 1)

            drain_store(c, 0)

            @pl.when(c + 2 < _NCH)
            def _():
                fire(c + 2, 0)

            drain_store(c + 1, 1)
            return carry

        lax.fori_loop(0, _NCH // 2, step, 0)

    return k(pidx, tab2)


def _tc_head(eug2, eig2, eum2, eim2, par_u, par_i, w1a_t, w1b_t, b1r, wg, wh,
             bp11):
    BB = 2048

    def body(eug_r, eig_r, eum_r, eim_r, pu_r, pi_r, w1a_r, w1b_r, b1_r,
             wg_r, wh_r, bp_r, out_r):
        sel_u = pu_r[...] == 0
        sel_i = pi_r[...] == 0
        eug = jnp.where(sel_u, eug_r[:, :F], eug_r[:, F:])
        eum = jnp.where(sel_u, eum_r[:, :F], eum_r[:, F:])
        eig = jnp.where(sel_i, eig_r[:, :F], eig_r[:, F:])
        eim = jnp.where(sel_i, eim_r[:, :F], eim_r[:, F:])
        h = jnp.dot(eum, w1a_r[...], preferred_element_type=jnp.float32)
        h = h + jnp.dot(eim, w1b_r[...], preferred_element_type=jnp.float32)
        h = jnp.maximum(h + b1_r[...], 0.0)
        gmf = eug * eig
        logit = (jnp.sum(gmf * wg_r[...], axis=1, keepdims=True)
                 + jnp.sum(h * wh_r[...], axis=1, keepdims=True)
                 + bp_r[...])
        out_r[...] = 1.0 / (1.0 + jnp.exp(-logit))

    batch_spec = pl.BlockSpec((BB, 2 * F), lambda i: (i, 0))
    par_spec = pl.BlockSpec((BB, 1), lambda i: (i, 0))
    full_spec = pl.BlockSpec((F, F), lambda i: (0, 0))
    row_spec = pl.BlockSpec((1, F), lambda i: (0, 0))
    return pl.pallas_call(
        body,
        grid=(B // BB,),
        in_specs=[batch_spec, batch_spec, batch_spec, batch_spec,
                  par_spec, par_spec,
                  full_spec, full_spec, row_spec, row_spec, row_spec,
                  pl.BlockSpec((1, 1), lambda i: (0, 0))],
        out_specs=pl.BlockSpec((BB, 1), lambda i: (i, 0)),
        out_shape=jax.ShapeDtypeStruct((B, 1), jnp.float32),
    )(eug2, eig2, eum2, eim2, par_u, par_i, w1a_t, w1b_t, b1r, wg, wh, bp11)


def kernel(user, item, embed_user_GMF, embed_item_GMF, embed_user_MLP,
           embed_item_MLP, W1, b1, Wp, bp):
    user = user.astype(jnp.int32)
    item = item.astype(jnp.int32)
    pu = user >> 1
    pi = item >> 1
    par_u = (user & 1).reshape(B, 1)
    par_i = (item & 1).reshape(B, 1)
    eug2 = _sc_gather1(pu, embed_user_GMF.reshape(-1, 2 * F))
    eum2 = _sc_gather1(pu, embed_user_MLP.reshape(-1, 2 * F))
    eig2 = _sc_gather1(pi, embed_item_GMF.reshape(-1, 2 * F))
    eim2 = _sc_gather1(pi, embed_item_MLP.reshape(-1, 2 * F))
    w1a_t = W1[:, :F].T
    w1b_t = W1[:, F:].T
    b1r = b1.reshape(1, F)
    wg = Wp[0, :F].reshape(1, F)
    wh = Wp[0, F:].reshape(1, F)
    bp11 = bp.reshape(1, 1)
    out = _tc_head(eug2, eig2, eum2, eim2, par_u, par_i, w1a_t, w1b_t, b1r,
                   wg, wh, bp11)
    return out.reshape(B)


# speedup vs baseline: 8.1245x; 1.0600x over previous
import functools

import jax
import jax.numpy as jnp
from jax import lax
from jax.experimental import pallas as pl
from jax.experimental.pallas import tpu as pltpu
from jax.experimental.pallas import tpu_sc as plsc

B = 16384
F = 64
_NC = 2
_NS = 16
_NW = _NC * _NS
_BPW = B // _NW
_CH = 128
_NCH = _BPW // _CH


def _sc_gather1(pidx, tab2):
    mesh = plsc.VectorSubcoreMesh(core_axis_name="c", subcore_axis_name="s")

    @functools.partial(
        pl.kernel,
        mesh=mesh,
        out_type=jax.ShapeDtypeStruct((B, 2 * F), jnp.float32),
        scratch_types=[
            pltpu.VMEM((_CH,), jnp.int32),
            pltpu.VMEM((_CH,), jnp.int32),
            pltpu.VMEM((_CH, 2 * F), jnp.float32),
            pltpu.VMEM((_CH, 2 * F), jnp.float32),
            pltpu.SemaphoreType.DMA,
            pltpu.SemaphoreType.DMA,
        ],
    )
    def k(pidx_h, tab_h, out_h, idx0, idx1, buf0, buf1, sem0, sem1):
        wid = lax.axis_index("s") * _NC + lax.axis_index("c")
        base = wid * _BPW
        idxs = (idx0, idx1)
        bufs = (buf0, buf1)
        sems = (sem0, sem1)

        def fire(c, sp):
            off = base + c * _CH
            pltpu.sync_copy(pidx_h.at[pl.ds(off, _CH)], idxs[sp])
            pltpu.async_copy(tab_h.at[idxs[sp]], bufs[sp], sems[sp])

        def drain_store(c, sp):
            pltpu.make_async_copy(tab_h.at[idxs[sp]], bufs[sp],
                                  sems[sp]).wait()
            off = base + c * _CH
            pltpu.sync_copy(bufs[sp], out_h.at[pl.ds(off, _CH)])

        fire(0, 0)

        def step(t, carry):
            c = t * 2

            @pl.when(c + 1 < _NCH)
            def _():
                fire(c + 1, 1)

            drain_store(c, 0)

            @pl.when(c + 2 < _NCH)
            def _():
                fire(c + 2, 0)

            drain_store(c + 1, 1)
            return carry

        lax.fori_loop(0, _NCH // 2, step, 0)

    return k(pidx, tab2)


def _tc_head(eug2, eig2, eum2, eim2, w1a_t, w1b_t, b1r, wg, wh,
             bp11):
    BB = 2048

    def body(eug_r, eig_r, eum_r, eim_r, w1a_r, w1b_r, b1_r,
             wg_r, wh_r, bp_r, out_r):
        eug = eug_r[:, :F]
        eum = eum_r[:, :F]
        eig = eig_r[:, :F]
        eim = eim_r[:, :F]
        h = jnp.dot(eum, w1a_r[...], preferred_element_type=jnp.float32)
        h = h + jnp.dot(eim, w1b_r[...], preferred_element_type=jnp.float32)
        h = jnp.maximum(h + b1_r[...], 0.0)
        gmf = eug * eig
        logit = (jnp.sum(gmf * wg_r[...], axis=1, keepdims=True)
                 + jnp.sum(h * wh_r[...], axis=1, keepdims=True)
                 + bp_r[...])
        out_r[...] = 1.0 / (1.0 + jnp.exp(-logit))

    batch_spec = pl.BlockSpec((BB, 2 * F), lambda i: (i, 0))
    full_spec = pl.BlockSpec((F, F), lambda i: (0, 0))
    row_spec = pl.BlockSpec((1, F), lambda i: (0, 0))
    return pl.pallas_call(
        body,
        grid=(B // BB,),
        in_specs=[batch_spec, batch_spec, batch_spec, batch_spec,
                  full_spec, full_spec, row_spec, row_spec, row_spec,
                  pl.BlockSpec((1, 1), lambda i: (0, 0))],
        out_specs=pl.BlockSpec((BB, 1), lambda i: (i, 0)),
        out_shape=jax.ShapeDtypeStruct((B, 1), jnp.float32),
    )(eug2, eig2, eum2, eim2, w1a_t, w1b_t, b1r, wg, wh, bp11)


def kernel(user, item, embed_user_GMF, embed_item_GMF, embed_user_MLP,
           embed_item_MLP, W1, b1, Wp, bp):
    user = user.astype(jnp.int32)
    item = item.astype(jnp.int32)
    pad = ((0, 0), (0, F))
    eug2 = _sc_gather1(user, jnp.pad(embed_user_GMF, pad))
    eum2 = _sc_gather1(user, jnp.pad(embed_user_MLP, pad))
    eig2 = _sc_gather1(item, jnp.pad(embed_item_GMF, pad))
    eim2 = _sc_gather1(item, jnp.pad(embed_item_MLP, pad))
    w1a_t = W1[:, :F].T
    w1b_t = W1[:, F:].T
    b1r = b1.reshape(1, F)
    wg = Wp[0, :F].reshape(1, F)
    wh = Wp[0, F:].reshape(1, F)
    bp11 = bp.reshape(1, 1)
    out = _tc_head(eug2, eig2, eum2, eim2, w1a_t, w1b_t, b1r,
                   wg, wh, bp11)
    return out.reshape(B)


# speedup vs baseline: 8.1324x; 1.0010x over previous
import functools

import jax
import jax.numpy as jnp
from jax import lax
from jax.experimental import pallas as pl
from jax.experimental.pallas import tpu as pltpu
from jax.experimental.pallas import tpu_sc as plsc

B = 16384
F = 64
_NC = 2
_NS = 16
_NW = _NC * _NS
_BPW = B // _NW
_CH = 128
_NCH = _BPW // _CH


def _sc_gather1(pidx, tab2):
    mesh = plsc.VectorSubcoreMesh(core_axis_name="c", subcore_axis_name="s")

    @functools.partial(
        pl.kernel,
        mesh=mesh,
        out_type=jax.ShapeDtypeStruct((B, 2 * F), jnp.float32),
        scratch_types=[
            pltpu.VMEM((_CH,), jnp.int32),
            pltpu.VMEM((_CH,), jnp.int32),
            pltpu.VMEM((_CH, 2 * F), jnp.float32),
            pltpu.VMEM((_CH, 2 * F), jnp.float32),
            pltpu.SemaphoreType.DMA,
            pltpu.SemaphoreType.DMA,
        ],
    )
    def k(pidx_h, tab_h, out_h, idx0, idx1, buf0, buf1, sem0, sem1):
        wid = lax.axis_index("s") * _NC + lax.axis_index("c")
        base = wid * _BPW
        idxs = (idx0, idx1)
        bufs = (buf0, buf1)
        sems = (sem0, sem1)

        def fire(c, sp):
            off = base + c * _CH
            pltpu.sync_copy(pidx_h.at[pl.ds(off, _CH)], idxs[sp])
            pltpu.async_copy(tab_h.at[idxs[sp]], bufs[sp], sems[sp])

        def drain_store(c, sp):
            pltpu.make_async_copy(tab_h.at[idxs[sp]], bufs[sp],
                                  sems[sp]).wait()
            off = base + c * _CH
            pltpu.sync_copy(bufs[sp], out_h.at[pl.ds(off, _CH)])

        fire(0, 0)

        def step(t, carry):
            c = t * 2

            @pl.when(c + 1 < _NCH)
            def _():
                fire(c + 1, 1)

            drain_store(c, 0)

            @pl.when(c + 2 < _NCH)
            def _():
                fire(c + 2, 0)

            drain_store(c + 1, 1)
            return carry

        lax.fori_loop(0, _NCH // 2, step, 0)

    return k(pidx, tab2)


def _tc_head(eug2, eig2, eum2, eim2, w1a_t, w1b_t, b1r, wg, wh,
             bp11):
    BB = 2048

    def body(eug_r, eig_r, eum_r, eim_r, w1a_r, w1b_r, b1_r,
             wg_r, wh_r, bp_r, out_r):
        eug = eug_r[:, :F]
        eum = eum_r[:, :F]
        eig = eig_r[:, :F]
        eim = eim_r[:, :F]
        h = jnp.dot(eum, w1a_r[...], preferred_element_type=jnp.float32)
        h = h + jnp.dot(eim, w1b_r[...], preferred_element_type=jnp.float32)
        h = jnp.maximum(h + b1_r[...], 0.0)
        gmf = eug * eig
        logit = (jnp.sum(gmf * wg_r[...], axis=1, keepdims=True)
                 + jnp.sum(h * wh_r[...], axis=1, keepdims=True)
                 + bp_r[...])
        out_r[...] = 1.0 / (1.0 + jnp.exp(-logit))

    batch_spec = pl.BlockSpec((BB, 2 * F), lambda i: (i, 0))
    full_spec = pl.BlockSpec((F, F), lambda i: (0, 0))
    row_spec = pl.BlockSpec((1, F), lambda i: (0, 0))
    return pl.pallas_call(
        body,
        grid=(B // BB,),
        in_specs=[batch_spec, batch_spec, batch_spec, batch_spec,
                  full_spec, full_spec, row_spec, row_spec, row_spec,
                  pl.BlockSpec((1, 1), lambda i: (0, 0))],
        out_specs=pl.BlockSpec((BB, 1), lambda i: (i, 0)),
        out_shape=jax.ShapeDtypeStruct((B, 1), jnp.float32),
    )(eug2, eig2, eum2, eim2, w1a_t, w1b_t, b1r, wg, wh, bp11)


def kernel(user, item, embed_user_GMF, embed_item_GMF, embed_user_MLP,
           embed_item_MLP, W1, b1, Wp, bp):
    user = user.astype(jnp.int32)
    item = item.astype(jnp.int32)
    pad = ((0, 0), (0, F))
    eig2 = _sc_gather1(item, jnp.pad(embed_item_GMF, pad))
    eim2 = _sc_gather1(item, jnp.pad(embed_item_MLP, pad))
    eug2 = _sc_gather1(user, jnp.pad(embed_user_GMF, pad))
    eum2 = _sc_gather1(user, jnp.pad(embed_user_MLP, pad))
    w1a_t = W1[:, :F].T
    w1b_t = W1[:, F:].T
    b1r = b1.reshape(1, F)
    wg = Wp[0, :F].reshape(1, F)
    wh = Wp[0, F:].reshape(1, F)
    bp11 = bp.reshape(1, 1)
    out = _tc_head(eug2, eig2, eum2, eim2, w1a_t, w1b_t, b1r,
                   wg, wh, bp11)
    return out.reshape(B)
